# baseline (speedup 1.0000x reference)
import functools

import jax
import jax.numpy as jnp
import numpy as np
from jax import lax
from jax.experimental import pallas as pl
from jax.experimental.pallas import tpu as pltpu
from jax.experimental.pallas import tpu_sc as plsc

_NUM_LODS = 16
_TABLE = 4096
_N = 262144
_HIDDEN = 16
_OUT = 3
_MIN_RES, _MAX_RES = 16, 512

_bf = np.exp((np.log(_MAX_RES) - np.log(_MIN_RES)) / (_NUM_LODS - 1))
_RES = [int(np.floor(_MIN_RES * (_bf ** l))) for l in range(_NUM_LODS)]

_NC, _NS, _L = 2, 16, 16
_NW = _NC * _NS
_NH = _N
_PPW = _NH // _NW
_SUB = 1024

_mesh = plsc.VectorSubcoreMesh(core_axis_name="c", subcore_axis_name="s")


def _make_sc(glob_off):
    @functools.partial(
        pl.kernel,
        mesh=_mesh,
        compiler_params=pltpu.CompilerParams(needs_layout_passes=False),
        out_type=jax.ShapeDtypeStruct((_NUM_LODS, _NH), jnp.float32),
        scratch_types=(
            [pltpu.VMEM((_TABLE,), jnp.float32) for _ in range(_NUM_LODS)]
            + [
                pltpu.VMEM((_PPW,), jnp.float32),
                pltpu.VMEM((_PPW,), jnp.float32),
                pltpu.VMEM((2, _NUM_LODS, _SUB), jnp.float32),
                pltpu.SemaphoreType.DMA,
                pltpu.SemaphoreType.DMA,
            ]
        ),
    )
    def sc_latents(xt_hbm, tab_hbm, out_hbm, *refs):
        tabs = refs[:_NUM_LODS]
        x0_v, x1_v, fe_v, sem0, sem1 = refs[_NUM_LODS:]
        wid = lax.axis_index("s") * _NC + lax.axis_index("c")
        base = wid * _PPW
        for l in range(_NUM_LODS):
            pltpu.sync_copy(tab_hbm.at[pl.ds(l * _TABLE, _TABLE)], tabs[l])
        pltpu.sync_copy(xt_hbm.at[0, pl.ds(glob_off + base, _PPW)], x0_v)
        pltpu.sync_copy(xt_hbm.at[1, pl.ds(glob_off + base, _PPW)], x1_v)
        kmul = jnp.int32(2654435761 - (1 << 32))
        mask = jnp.int32(_TABLE - 1)
        sems = (sem0, sem1)

        def outer(b, carry):
            for k in (0, 1):
                sboff = b * (2 * _SUB) + k * _SUB

                @pl.when(b > 0)
                def _drain():
                    pltpu.make_async_copy(
                        out_hbm.at[:, pl.ds(base, _SUB)], fe_v.at[k], sems[k]
                    ).wait()

                @plsc.parallel_loop(0, _SUB // _L, unroll=4)
                def body(i, sboff=sboff, k=k):
                    xv = x0_v[pl.ds(sboff + i * _L, _L)]
                    yv = x1_v[pl.ds(sboff + i * _L, _L)]
                    for l in range(_NUM_LODS):
                        r = float(_RES[l])
                        px = xv * r
                        py = yv * r
                        ix = px.astype(jnp.int32)
                        iy = py.astype(jnp.int32)
                        fx = px - ix.astype(jnp.float32)
                        fy = py - iy.astype(jnp.float32)
                        hy0 = iy * kmul
                        hy1 = hy0 + kmul
                        ix1 = ix + jnp.int32(1)
                        i00 = (ix ^ hy0) & mask
                        i10 = (ix1 ^ hy0) & mask
                        i01 = (ix ^ hy1) & mask
                        i11 = (ix1 ^ hy1) & mask
                        f00 = plsc.load_gather(tabs[l], [i00])
                        f10 = plsc.load_gather(tabs[l], [i10])
                        f01 = plsc.load_gather(tabs[l], [i01])
                        f11 = plsc.load_gather(tabs[l], [i11])
                        a = f00 + fx * (f10 - f00)
                        b2_ = f01 + fx * (f11 - f01)
                        fe_v[k, l, pl.ds(i * _L, _L)] = a + fy * (b2_ - a)

                pltpu.async_copy(
                    fe_v.at[k], out_hbm.at[:, pl.ds(base + sboff, _SUB)],
                    sems[k])
            return carry

        lax.fori_loop(0, _PPW // (2 * _SUB), outer, 0)
        for k in (0, 1):
            pltpu.make_async_copy(
                out_hbm.at[:, pl.ds(base, _SUB)], fe_v.at[k], sems[k]
            ).wait()

    return sc_latents


_sc_half = (_make_sc(0),)


def _mlp_body(f_ref, w1_ref, b1_ref, w2_ref, b2_ref, o_ref):
    f = f_ref[...]
    h = jnp.dot(w1_ref[...], f, preferred_element_type=jnp.float32)
    h = jnp.maximum(h + b1_ref[...], 0.0)
    g = jnp.dot(w2_ref[...], h, preferred_element_type=jnp.float32)
    g = g + b2_ref[...]
    o_ref[...] = 1.0 / (1.0 + jnp.exp(-g))


def _mlp(lat_t, w1t, b1t, w2t, b2t):
    bn = 16384
    return pl.pallas_call(
        _mlp_body,
        grid=(_NH // bn,),
        in_specs=[
            pl.BlockSpec((_NUM_LODS, bn), lambda i: (0, i)),
            pl.BlockSpec((_HIDDEN, _NUM_LODS), lambda i: (0, 0)),
            pl.BlockSpec((_HIDDEN, 1), lambda i: (0, 0)),
            pl.BlockSpec((_OUT, _HIDDEN), lambda i: (0, 0)),
            pl.BlockSpec((_OUT, 1), lambda i: (0, 0)),
        ],
        out_specs=pl.BlockSpec((_OUT, bn), lambda i: (0, i)),
        out_shape=jax.ShapeDtypeStruct((_OUT, _NH), jnp.float32),
    )(lat_t, w1t, b1t, w2t, b2t)


def kernel(x, codebooks, dec_w, dec_b, w1, b1, w2, b2):
    xt = x.T
    tab = codebooks.reshape(_NUM_LODS * _TABLE)
    w1t = (w1 * dec_w[:, None]).T
    b1t = (b1 + dec_b @ w1).reshape(_HIDDEN, 1)
    w2t = w2.T
    b2t = b2.reshape(_OUT, 1)

    halves = [
        _mlp(_sc_half[h](xt, tab), w1t, b1t, w2t, b2t) for h in range(1)
    ]
    out_t = jnp.concatenate(halves, axis=1)
    return out_t.T


# speedup vs baseline: 1.0498x; 1.0498x over previous
import functools

import jax
import jax.numpy as jnp
import numpy as np
from jax import lax
from jax.experimental import pallas as pl
from jax.experimental.pallas import tpu as pltpu
from jax.experimental.pallas import tpu_sc as plsc

_NUM_LODS = 16
_TABLE = 4096
_N = 262144
_HIDDEN = 16
_OUT = 3
_MIN_RES, _MAX_RES = 16, 512

_bf = np.exp((np.log(_MAX_RES) - np.log(_MIN_RES)) / (_NUM_LODS - 1))
_RES = [int(np.floor(_MIN_RES * (_bf ** l))) for l in range(_NUM_LODS)]

_NC, _NS, _L = 2, 16, 16
_NW = _NC * _NS
_NH = _N
_PPW = _NH // _NW
_SUB = 1024

_mesh = plsc.VectorSubcoreMesh(core_axis_name="c", subcore_axis_name="s")


def _make_sc(glob_off):
    @functools.partial(
        pl.kernel,
        mesh=_mesh,
        compiler_params=pltpu.CompilerParams(needs_layout_passes=False),
        out_type=jax.ShapeDtypeStruct((_NUM_LODS, _NH), jnp.float32),
        scratch_types=(
            [pltpu.VMEM((_TABLE,), jnp.float32) for _ in range(_NUM_LODS)]
            + [
                pltpu.VMEM((_PPW,), jnp.float32),
                pltpu.VMEM((_PPW,), jnp.float32),
                pltpu.VMEM((2, _NUM_LODS, _SUB), jnp.float32),
                pltpu.SemaphoreType.DMA,
                pltpu.SemaphoreType.DMA,
            ]
        ),
    )
    def sc_latents(xt_hbm, tab_hbm, out_hbm, *refs):
        tabs = refs[:_NUM_LODS]
        x0_v, x1_v, fe_v, sem0, sem1 = refs[_NUM_LODS:]
        wid = lax.axis_index("s") * _NC + lax.axis_index("c")
        base = wid * _PPW
        for l in range(_NUM_LODS):
            pltpu.sync_copy(tab_hbm.at[pl.ds(l * _TABLE, _TABLE)], tabs[l])
        pltpu.sync_copy(xt_hbm.at[0, pl.ds(glob_off + base, _PPW)], x0_v)
        pltpu.sync_copy(xt_hbm.at[1, pl.ds(glob_off + base, _PPW)], x1_v)
        kmul = jnp.int32(2654435761 - (1 << 32))
        mask = jnp.int32(_TABLE - 1)
        sems = (sem0, sem1)

        def outer(b, carry):
            for k in (0, 1):
                sboff = b * (2 * _SUB) + k * _SUB

                @pl.when(b > 0)
                def _drain():
                    pltpu.make_async_copy(
                        out_hbm.at[:, pl.ds(base, _SUB)], fe_v.at[k], sems[k]
                    ).wait()

                @plsc.parallel_loop(0, _SUB // _L, unroll=4)
                def body(i, sboff=sboff, k=k):
                    xv = x0_v[pl.ds(sboff + i * _L, _L)]
                    yv = x1_v[pl.ds(sboff + i * _L, _L)]
                    for l in range(_NUM_LODS):
                        r = float(_RES[l])
                        px = xv * r
                        py = yv * r
                        ix = px.astype(jnp.int32)
                        iy = py.astype(jnp.int32)
                        fx = px - ix.astype(jnp.float32)
                        fy = py - iy.astype(jnp.float32)
                        hy0 = iy * kmul
                        hy1 = hy0 + kmul
                        ix1 = ix + jnp.int32(1)
                        i00 = (ix ^ hy0) & mask
                        i10 = (ix1 ^ hy0) & mask
                        i01 = (ix ^ hy1) & mask
                        i11 = (ix1 ^ hy1) & mask
                        f00 = plsc.load_gather(tabs[l], [i00])
                        f10 = plsc.load_gather(tabs[l], [i10])
                        f01 = plsc.load_gather(tabs[l], [i01])
                        f11 = plsc.load_gather(tabs[l], [i11])
                        a = f00 + fx * (f10 - f00)
                        b2_ = f01 + fx * (f11 - f01)
                        fe_v[k, l, pl.ds(i * _L, _L)] = a + fy * (b2_ - a)

                pltpu.async_copy(
                    fe_v.at[k], out_hbm.at[:, pl.ds(base + sboff, _SUB)],
                    sems[k])
            return carry

        lax.fori_loop(0, _PPW // (2 * _SUB), outer, 0)
        for k in (0, 1):
            pltpu.make_async_copy(
                out_hbm.at[:, pl.ds(base, _SUB)], fe_v.at[k], sems[k]
            ).wait()

    return sc_latents


_sc_half = (_make_sc(0),)


def _mlp_body(f_ref, w1_ref, b1_ref, w2_ref, b2_ref, o_ref):
    f = f_ref[...]
    h = jnp.dot(w1_ref[...], f, preferred_element_type=jnp.float32)
    h = jnp.maximum(h + b1_ref[...], 0.0)
    g = jnp.dot(w2_ref[...], h, preferred_element_type=jnp.float32)
    g = g + b2_ref[...]
    o_ref[...] = 1.0 / (1.0 + jnp.exp(-g))


def _mlp(lat_t, w1t, b1t, w2t, b2t):
    bn = 65536
    return pl.pallas_call(
        _mlp_body,
        grid=(_NH // bn,),
        in_specs=[
            pl.BlockSpec((_NUM_LODS, bn), lambda i: (0, i)),
            pl.BlockSpec((_HIDDEN, _NUM_LODS), lambda i: (0, 0)),
            pl.BlockSpec((_HIDDEN, 1), lambda i: (0, 0)),
            pl.BlockSpec((_OUT, _HIDDEN), lambda i: (0, 0)),
            pl.BlockSpec((_OUT, 1), lambda i: (0, 0)),
        ],
        out_specs=pl.BlockSpec((_OUT, bn), lambda i: (0, i)),
        out_shape=jax.ShapeDtypeStruct((_OUT, _NH), jnp.float32),
    )(lat_t, w1t, b1t, w2t, b2t)


def kernel(x, codebooks, dec_w, dec_b, w1, b1, w2, b2):
    xt = x.T
    tab = codebooks.reshape(_NUM_LODS * _TABLE)
    w1t = (w1 * dec_w[:, None]).T
    b1t = (b1 + dec_b @ w1).reshape(_HIDDEN, 1)
    w2t = w2.T
    b2t = b2.reshape(_OUT, 1)

    halves = [
        _mlp(_sc_half[h](xt, tab), w1t, b1t, w2t, b2t) for h in range(1)
    ]
    out_t = jnp.concatenate(halves, axis=1)
    return out_t.T


# speedup vs baseline: 1.0525x; 1.0026x over previous
import functools

import jax
import jax.numpy as jnp
import numpy as np
from jax import lax
from jax.experimental import pallas as pl
from jax.experimental.pallas import tpu as pltpu
from jax.experimental.pallas import tpu_sc as plsc

_NUM_LODS = 16
_TABLE = 4096
_N = 262144
_HIDDEN = 16
_OUT = 3
_MIN_RES, _MAX_RES = 16, 512

_bf = np.exp((np.log(_MAX_RES) - np.log(_MIN_RES)) / (_NUM_LODS - 1))
_RES = [int(np.floor(_MIN_RES * (_bf ** l))) for l in range(_NUM_LODS)]

_NC, _NS, _L = 2, 16, 16
_NW = _NC * _NS
_NH = _N
_PPW = _NH // _NW
_SUB = 1024

_mesh = plsc.VectorSubcoreMesh(core_axis_name="c", subcore_axis_name="s")


def _make_sc(glob_off):
    @functools.partial(
        pl.kernel,
        mesh=_mesh,
        compiler_params=pltpu.CompilerParams(needs_layout_passes=False),
        out_type=jax.ShapeDtypeStruct((_NUM_LODS, _NH), jnp.float32),
        scratch_types=(
            [pltpu.VMEM((_TABLE,), jnp.float32) for _ in range(_NUM_LODS)]
            + [
                pltpu.VMEM((_PPW,), jnp.float32),
                pltpu.VMEM((_PPW,), jnp.float32),
                pltpu.VMEM((2, _NUM_LODS, _SUB), jnp.float32),
                pltpu.SemaphoreType.DMA,
                pltpu.SemaphoreType.DMA,
            ]
        ),
    )
    def sc_latents(xt_hbm, tab_hbm, out_hbm, *refs):
        tabs = refs[:_NUM_LODS]
        x0_v, x1_v, fe_v, sem0, sem1 = refs[_NUM_LODS:]
        wid = lax.axis_index("s") * _NC + lax.axis_index("c")
        base = wid * _PPW
        for l in range(_NUM_LODS):
            pltpu.sync_copy(tab_hbm.at[pl.ds(l * _TABLE, _TABLE)], tabs[l])
        pltpu.sync_copy(xt_hbm.at[0, pl.ds(glob_off + base, _PPW)], x0_v)
        pltpu.sync_copy(xt_hbm.at[1, pl.ds(glob_off + base, _PPW)], x1_v)
        kmul = jnp.int32(2654435761 - (1 << 32))
        mask = jnp.int32(_TABLE - 1)
        sems = (sem0, sem1)

        def outer(b, carry):
            for k in (0, 1):
                sboff = b * (2 * _SUB) + k * _SUB

                @pl.when(b > 0)
                def _drain():
                    pltpu.make_async_copy(
                        out_hbm.at[:, pl.ds(base, _SUB)], fe_v.at[k], sems[k]
                    ).wait()

                @plsc.parallel_loop(0, _SUB // _L, unroll=4)
                def body(i, sboff=sboff, k=k):
                    xv = x0_v[pl.ds(sboff + i * _L, _L)]
                    yv = x1_v[pl.ds(sboff + i * _L, _L)]
                    for l in range(_NUM_LODS):
                        r = float(_RES[l])
                        px = xv * r
                        py = yv * r
                        ix = px.astype(jnp.int32)
                        iy = py.astype(jnp.int32)
                        fx = px - ix.astype(jnp.float32)
                        fy = py - iy.astype(jnp.float32)
                        hy0 = iy * kmul
                        hy1 = hy0 + kmul
                        ix1 = ix + jnp.int32(1)
                        i00 = (ix ^ hy0) & mask
                        i10 = (ix1 ^ hy0) & mask
                        i01 = (ix ^ hy1) & mask
                        i11 = (ix1 ^ hy1) & mask
                        f00 = plsc.load_gather(tabs[l], [i00])
                        f10 = plsc.load_gather(tabs[l], [i10])
                        f01 = plsc.load_gather(tabs[l], [i01])
                        f11 = plsc.load_gather(tabs[l], [i11])
                        a = f00 + fx * (f10 - f00)
                        b2_ = f01 + fx * (f11 - f01)
                        fe_v[k, l, pl.ds(i * _L, _L)] = a + fy * (b2_ - a)

                pltpu.async_copy(
                    fe_v.at[k], out_hbm.at[:, pl.ds(base + sboff, _SUB)],
                    sems[k])
            return carry

        lax.fori_loop(0, _PPW // (2 * _SUB), outer, 0)
        for k in (0, 1):
            pltpu.make_async_copy(
                out_hbm.at[:, pl.ds(base, _SUB)], fe_v.at[k], sems[k]
            ).wait()

    return sc_latents


_sc_half = (_make_sc(0),)


def _mlp_body(f_ref, w1_ref, b1_ref, w2_ref, b2_ref, o_ref):
    f = f_ref[...]
    h = jnp.dot(w1_ref[...], f, preferred_element_type=jnp.float32)
    h = jnp.maximum(h + b1_ref[...], 0.0)
    g = jnp.dot(w2_ref[...], h, preferred_element_type=jnp.float32)
    g = g + b2_ref[...]
    o_ref[...] = 1.0 / (1.0 + jnp.exp(-g))


def _mlp(lat_t, w1t, b1t, w2t, b2t):
    bn = 131072
    return pl.pallas_call(
        _mlp_body,
        grid=(_NH // bn,),
        in_specs=[
            pl.BlockSpec((_NUM_LODS, bn), lambda i: (0, i)),
            pl.BlockSpec((_HIDDEN, _NUM_LODS), lambda i: (0, 0)),
            pl.BlockSpec((_HIDDEN, 1), lambda i: (0, 0)),
            pl.BlockSpec((_OUT, _HIDDEN), lambda i: (0, 0)),
            pl.BlockSpec((_OUT, 1), lambda i: (0, 0)),
        ],
        out_specs=pl.BlockSpec((_OUT, bn), lambda i: (0, i)),
        out_shape=jax.ShapeDtypeStruct((_OUT, _NH), jnp.float32),
    )(lat_t, w1t, b1t, w2t, b2t)


def kernel(x, codebooks, dec_w, dec_b, w1, b1, w2, b2):
    xt = x.T
    tab = codebooks.reshape(_NUM_LODS * _TABLE)
    w1t = (w1 * dec_w[:, None]).T
    b1t = (b1 + dec_b @ w1).reshape(_HIDDEN, 1)
    w2t = w2.T
    b2t = b2.reshape(_OUT, 1)

    halves = [
        _mlp(_sc_half[h](xt, tab), w1t, b1t, w2t, b2t) for h in range(1)
    ]
    out_t = jnp.concatenate(halves, axis=1)
    return out_t.T


# speedup vs baseline: 1.1367x; 1.0800x over previous
import functools

import jax
import jax.numpy as jnp
import numpy as np
from jax import lax
from jax.experimental import pallas as pl
from jax.experimental.pallas import tpu as pltpu
from jax.experimental.pallas import tpu_sc as plsc

_NUM_LODS = 16
_TABLE = 4096
_N = 262144
_HIDDEN = 16
_OUT = 3
_MIN_RES, _MAX_RES = 16, 512

_bf = np.exp((np.log(_MAX_RES) - np.log(_MIN_RES)) / (_NUM_LODS - 1))
_RES = [int(np.floor(_MIN_RES * (_bf ** l))) for l in range(_NUM_LODS)]

_NC, _NS, _L = 2, 16, 16
_NW = _NC * _NS
_NH = _N
_PPW = _NH // _NW
_SUB = 1024

_mesh = plsc.VectorSubcoreMesh(core_axis_name="c", subcore_axis_name="s")


def _make_sc(glob_off):
    @functools.partial(
        pl.kernel,
        mesh=_mesh,
        compiler_params=pltpu.CompilerParams(needs_layout_passes=False),
        out_type=jax.ShapeDtypeStruct((_NUM_LODS // 2, _NH), jnp.float32),
        scratch_types=(
            [pltpu.VMEM((_TABLE,), jnp.float32) for _ in range(_NUM_LODS)]
            + [
                pltpu.VMEM((_PPW,), jnp.float32),
                pltpu.VMEM((_PPW,), jnp.float32),
                pltpu.VMEM((2, _NUM_LODS // 2, _SUB), jnp.float32),
                pltpu.SemaphoreType.DMA,
                pltpu.SemaphoreType.DMA,
            ]
        ),
    )
    def sc_latents(xt_hbm, tab_hbm, out_hbm, *refs):
        tabs = refs[:_NUM_LODS]
        x0_v, x1_v, fe_v, sem0, sem1 = refs[_NUM_LODS:]
        wid = lax.axis_index("s") * _NC + lax.axis_index("c")
        base = wid * _PPW
        for l in range(_NUM_LODS):
            pltpu.sync_copy(tab_hbm.at[pl.ds(l * _TABLE, _TABLE)], tabs[l])
        pltpu.sync_copy(xt_hbm.at[0, pl.ds(glob_off + base, _PPW)], x0_v)
        pltpu.sync_copy(xt_hbm.at[1, pl.ds(glob_off + base, _PPW)], x1_v)
        kmul = jnp.int32(2654435761 - (1 << 32))
        mask = jnp.int32(_TABLE - 1)
        sems = (sem0, sem1)

        def outer(b, carry):
            for k in (0, 1):
                sboff = b * (2 * _SUB) + k * _SUB

                @pl.when(b > 0)
                def _drain():
                    pltpu.make_async_copy(
                        out_hbm.at[:, pl.ds(base, _SUB)], fe_v.at[k], sems[k]
                    ).wait()

                @plsc.parallel_loop(0, _SUB // _L, unroll=4)
                def body(i, sboff=sboff, k=k):
                    xv = x0_v[pl.ds(sboff + i * _L, _L)]
                    yv = x1_v[pl.ds(sboff + i * _L, _L)]
                    for j in range(_NUM_LODS // 2):
                        pair = []
                        for l in (2 * j, 2 * j + 1):
                            r = float(_RES[l])
                            px = xv * r
                            py = yv * r
                            ix = px.astype(jnp.int32)
                            iy = py.astype(jnp.int32)
                            fx = px - ix.astype(jnp.float32)
                            fy = py - iy.astype(jnp.float32)
                            hy0 = iy * kmul
                            hy1 = hy0 + kmul
                            ix1 = ix + jnp.int32(1)
                            i00 = (ix ^ hy0) & mask
                            i10 = (ix1 ^ hy0) & mask
                            i01 = (ix ^ hy1) & mask
                            i11 = (ix1 ^ hy1) & mask
                            f00 = plsc.load_gather(tabs[l], [i00])
                            f10 = plsc.load_gather(tabs[l], [i10])
                            f01 = plsc.load_gather(tabs[l], [i01])
                            f11 = plsc.load_gather(tabs[l], [i11])
                            a = f00 + fx * (f10 - f00)
                            b2_ = f01 + fx * (f11 - f01)
                            pair.append(a + fy * (b2_ - a))
                        packed = plsc.bitcast(
                            plsc.pack(pair[0], pair[1],
                                      format=plsc.PackFormat.INTERLEAVED),
                            jnp.float32)
                        fe_v[k, j, pl.ds(i * _L, _L)] = packed

                pltpu.async_copy(
                    fe_v.at[k], out_hbm.at[:, pl.ds(base + sboff, _SUB)],
                    sems[k])
            return carry

        lax.fori_loop(0, _PPW // (2 * _SUB), outer, 0)
        for k in (0, 1):
            pltpu.make_async_copy(
                out_hbm.at[:, pl.ds(base, _SUB)], fe_v.at[k], sems[k]
            ).wait()

    return sc_latents


_sc_half = (_make_sc(0),)


def _mlp_body(f_ref, w1_ref, b1_ref, w2_ref, b2_ref, o_ref):
    f = pltpu.bitcast(f_ref[...], jnp.bfloat16).astype(jnp.float32)
    h = jnp.dot(w1_ref[...], f, preferred_element_type=jnp.float32)
    h = jnp.maximum(h + b1_ref[...], 0.0)
    g = jnp.dot(w2_ref[...], h, preferred_element_type=jnp.float32)
    g = g + b2_ref[...]
    o_ref[...] = 1.0 / (1.0 + jnp.exp(-g))


def _mlp(lat_t, w1t, b1t, w2t, b2t):
    bn = 131072
    return pl.pallas_call(
        _mlp_body,
        grid=(_NH // bn,),
        in_specs=[
            pl.BlockSpec((_NUM_LODS // 2, bn), lambda i: (0, i)),
            pl.BlockSpec((_HIDDEN, _NUM_LODS), lambda i: (0, 0)),
            pl.BlockSpec((_HIDDEN, 1), lambda i: (0, 0)),
            pl.BlockSpec((_OUT, _HIDDEN), lambda i: (0, 0)),
            pl.BlockSpec((_OUT, 1), lambda i: (0, 0)),
        ],
        out_specs=pl.BlockSpec((_OUT, bn), lambda i: (0, i)),
        out_shape=jax.ShapeDtypeStruct((_OUT, _NH), jnp.float32),
    )(lat_t, w1t, b1t, w2t, b2t)


def kernel(x, codebooks, dec_w, dec_b, w1, b1, w2, b2):
    xt = x.T
    tab = codebooks.reshape(_NUM_LODS * _TABLE)
    w1t = (w1 * dec_w[:, None]).T
    b1t = (b1 + dec_b @ w1).reshape(_HIDDEN, 1)
    w2t = w2.T
    b2t = b2.reshape(_OUT, 1)

    halves = [
        _mlp(_sc_half[h](xt, tab), w1t, b1t, w2t, b2t) for h in range(1)
    ]
    out_t = jnp.concatenate(halves, axis=1)
    return out_t.T
